# baseline (speedup 1.0000x reference)
import functools

import jax
import jax.numpy as jnp
from jax import lax
from jax.experimental import pallas as pl
from jax.experimental.pallas import tpu as pltpu
from jax.experimental.pallas import tpu_sc as plsc

E = 16
EP1 = E + 1
D = 768
F = 256
T = 2048
LANES = 128
NT = T // LANES + EP1
NTOT = NT * LANES
TRASH = T
NC, NS = 2, 16


def _router_body(x_ref, wr_ref, mask_ref,
                 logits_ref, sel_ref, ids_ref, gs_ref, te_ref):
    x = x_ref[...]
    lm = jnp.dot(x, wr_ref[...], preferred_element_type=jnp.float32)
    lm = lm * mask_ref[...]
    col = lax.broadcasted_iota(jnp.int32, (T, LANES), 1)
    valid = col < EP1
    lmask = jnp.where(valid, lm, jnp.float32(-1e30))
    rowmax = jnp.max(lmask, axis=1, keepdims=True)
    ex = jnp.where(valid, jnp.exp(lmask - rowmax), 0.0)
    probs = ex / jnp.sum(ex, axis=1, keepdims=True)
    maxp = jnp.max(probs, axis=1, keepdims=True)
    sel = jnp.min(jnp.where(probs == maxp, col, LANES), axis=1, keepdims=True)
    gate = jnp.where(sel == E, 0.0, maxp)
    ohf = (col == sel).astype(jnp.float32)

    r = lax.broadcasted_iota(jnp.int32, (LANES, LANES), 0)
    c = lax.broadcasted_iota(jnp.int32, (LANES, LANES), 1)
    tril = (r >= c).astype(jnp.float32)
    carry = jnp.zeros((1, LANES), jnp.float32)
    incl_rows = []
    for b in range(T // LANES):
        blk = ohf[b * LANES:(b + 1) * LANES, :]
        incl_rows.append(
            jnp.dot(tril, blk, preferred_element_type=jnp.float32,
                    precision=lax.Precision.HIGHEST) + carry)
        carry = carry + jnp.sum(blk, axis=0, keepdims=True)
    incl = jnp.concatenate(incl_rows, axis=0)
    counts = carry

    lane = lax.broadcasted_iota(jnp.int32, (1, LANES), 1)
    pcounts = jnp.where(lane < EP1,
                        jnp.ceil(counts / float(LANES)) * float(LANES), 0.0)
    sut = (r < c).astype(jnp.float32)
    offs = jnp.dot(pcounts, sut, preferred_element_type=jnp.float32,
                   precision=lax.Precision.HIGHEST)
    rank = jnp.sum(incl * ohf, axis=1, keepdims=True) - 1.0
    base = jnp.sum(offs * ohf, axis=1, keepdims=True)
    pos = (base + rank).astype(jnp.int32)

    ends = offs + pcounts
    rowj = r.astype(jnp.float32) * float(LANES)
    filled = (jnp.broadcast_to(ends, (LANES, LANES)) <= rowj) & (c < EP1)
    te = jnp.sum(filled.astype(jnp.int32), axis=1, keepdims=True)
    te = jnp.minimum(te, E - 1)

    posf = pos
    tvec = lax.broadcasted_iota(jnp.int32, (T, 1), 0).astype(jnp.float32)
    ones = jnp.ones((T, 1), jnp.float32)
    slot = lax.broadcasted_iota(jnp.int32, (1, LANES), 1)
    for j in range(NT):
        ind = (posf == (slot + j * LANES)).astype(jnp.float32)
        idsj = lax.dot_general(ind, tvec, (((0,), (0,)), ((), ())),
                               preferred_element_type=jnp.float32,
                               precision=lax.Precision.HIGHEST)
        cov = lax.dot_general(ind, ones, (((0,), (0,)), ((), ())),
                              preferred_element_type=jnp.float32,
                              precision=lax.Precision.HIGHEST)
        gsj = lax.dot_general(ind, gate, (((0,), (0,)), ((), ())),
                              preferred_element_type=jnp.float32,
                              precision=lax.Precision.HIGHEST)
        ids_ref[j * LANES:(j + 1) * LANES, :] = (
            idsj + float(TRASH) * (1.0 - cov)).astype(jnp.int32)
        gs_ref[j * LANES:(j + 1) * LANES, :] = gsj

    logits_ref[...] = lm
    sel_ref[...] = sel
    te_ref[...] = te


def _gather_body(xpad_hbm, ids_hbm, xs_hbm, idxv, rows, sem):
    wid = lax.axis_index("s") * NC + lax.axis_index("c")

    def do_chunk(chunk):
        pltpu.sync_copy(ids_hbm.at[pl.ds(chunk * LANES, LANES)], idxv)
        pltpu.async_copy(xpad_hbm.at[idxv], rows, sem).wait()
        pltpu.sync_copy(rows, xs_hbm.at[pl.ds(chunk * LANES, LANES)])

    do_chunk(wid)

    @pl.when(wid == 0)
    def _():
        do_chunk(jnp.int32(NT - 1))


def _scatter_body(ys_hbm, ids_hbm, out_hbm, idxv, rows, sem):
    wid = lax.axis_index("s") * NC + lax.axis_index("c")

    def do_chunk(chunk):
        pltpu.sync_copy(ids_hbm.at[pl.ds(chunk * LANES, LANES)], idxv)
        pltpu.sync_copy(ys_hbm.at[pl.ds(chunk * LANES, LANES)], rows)
        pltpu.async_copy(rows, out_hbm.at[idxv], sem).wait()

    do_chunk(wid)

    @pl.when(wid == 0)
    def _():
        do_chunk(jnp.int32(NT - 1))


def _mlp_body(te_ref, x_ref, w1_ref, w2_ref, g_ref, y_ref):
    x = x_ref[...]
    h = jax.nn.gelu(jnp.dot(x, w1_ref[0], preferred_element_type=jnp.float32))
    y = jnp.dot(h, w2_ref[0], preferred_element_type=jnp.float32)
    y_ref[...] = y * g_ref[0]


def kernel(inputs, masks, W_router, W1, W2):
    x = inputs.reshape(T, D)
    wr_pad = jnp.zeros((D, LANES), jnp.float32).at[:, :EP1].set(W_router)
    masks_pad = jnp.concatenate(
        [masks, jnp.ones((T, 1), masks.dtype),
         jnp.zeros((T, LANES - EP1), masks.dtype)], axis=1)
    x_pad = jnp.concatenate([x, jnp.zeros((1, D), x.dtype)], axis=0)

    logits, sel, ids2d, gs2d, te = pl.pallas_call(
        _router_body,
        out_shape=(
            jax.ShapeDtypeStruct((T, LANES), jnp.float32),
            jax.ShapeDtypeStruct((T, 1), jnp.int32),
            jax.ShapeDtypeStruct((NTOT, 1), jnp.int32),
            jax.ShapeDtypeStruct((NTOT, 1), jnp.float32),
            jax.ShapeDtypeStruct((LANES, 1), jnp.int32),
        ),
    )(x, wr_pad, masks_pad)
    ids = ids2d.reshape(NTOT)

    mesh2 = plsc.VectorSubcoreMesh(core_axis_name="c", subcore_axis_name="s")
    gather = pl.kernel(
        _gather_body,
        out_type=jax.ShapeDtypeStruct((NTOT, D), jnp.float32),
        mesh=mesh2,
        scratch_types=[
            pltpu.VMEM((LANES,), jnp.int32),
            pltpu.VMEM((LANES, D), jnp.float32),
            pltpu.SemaphoreType.DMA,
        ],
    )
    xs = gather(x_pad, ids)

    grid_spec = pltpu.PrefetchScalarGridSpec(
        num_scalar_prefetch=1,
        grid=(NT,),
        in_specs=[
            pl.BlockSpec((LANES, D), lambda j, te: (j, 0)),
            pl.BlockSpec((1, D, F), lambda j, te: (te[j], 0, 0)),
            pl.BlockSpec((1, F, D), lambda j, te: (te[j], 0, 0)),
            pl.BlockSpec((1, LANES, 1), lambda j, te: (j, 0, 0)),
        ],
        out_specs=pl.BlockSpec((LANES, D), lambda j, te: (j, 0)),
    )
    ys = pl.pallas_call(
        _mlp_body,
        grid_spec=grid_spec,
        out_shape=jax.ShapeDtypeStruct((NTOT, D), jnp.float32),
    )(te.reshape(LANES), xs, W1, W2, gs2d.reshape(NT, LANES, 1))

    scatter = pl.kernel(
        _scatter_body,
        out_type=jax.ShapeDtypeStruct((T + 1, D), jnp.float32),
        mesh=mesh2,
        scratch_types=[
            pltpu.VMEM((LANES,), jnp.int32),
            pltpu.VMEM((LANES, D), jnp.float32),
            pltpu.SemaphoreType.DMA,
        ],
    )
    out = scatter(ys, ids)

    results = out[:T].reshape(inputs.shape)
    router_logits = logits[:, :EP1]
    return results, router_logits, sel


# speedup vs baseline: 1.3223x; 1.3223x over previous
import functools

import jax
import jax.numpy as jnp
from jax import lax
from jax.experimental import pallas as pl
from jax.experimental.pallas import tpu as pltpu
from jax.experimental.pallas import tpu_sc as plsc

E = 16
EP1 = E + 1
D = 768
F = 256
T = 2048
LANES = 128
NT = T // LANES + EP1
NTOT = NT * LANES
TRASH = T
NC, NS = 2, 16


def _router_body(x_ref, wr_ref, mask_ref,
                 logits_ref, sel_ref, ids_ref, gs_ref, te_ref):
    x = x_ref[...]
    lm = jnp.dot(x, wr_ref[...], preferred_element_type=jnp.float32)
    lm = lm * mask_ref[...]
    col = lax.broadcasted_iota(jnp.int32, (T, LANES), 1)
    valid = col < EP1
    lmask = jnp.where(valid, lm, jnp.float32(-1e30))
    rowmax = jnp.max(lmask, axis=1, keepdims=True)
    ex = jnp.where(valid, jnp.exp(lmask - rowmax), 0.0)
    probs = ex / jnp.sum(ex, axis=1, keepdims=True)
    maxp = jnp.max(probs, axis=1, keepdims=True)
    sel = jnp.min(jnp.where(probs == maxp, col, LANES), axis=1, keepdims=True)
    gate = jnp.where(sel == E, 0.0, maxp)
    ohf = (col == sel).astype(jnp.float32)

    r = lax.broadcasted_iota(jnp.int32, (LANES, LANES), 0)
    c = lax.broadcasted_iota(jnp.int32, (LANES, LANES), 1)
    tril = (r >= c).astype(jnp.float32)
    carry = jnp.zeros((1, LANES), jnp.float32)
    incl_rows = []
    for b in range(T // LANES):
        blk = ohf[b * LANES:(b + 1) * LANES, :]
        incl_rows.append(
            jnp.dot(tril, blk, preferred_element_type=jnp.float32) + carry)
        carry = carry + jnp.sum(blk, axis=0, keepdims=True)
    incl = jnp.concatenate(incl_rows, axis=0)
    counts = carry

    lane = lax.broadcasted_iota(jnp.int32, (1, LANES), 1)
    pcounts = jnp.where(lane < EP1,
                        jnp.ceil(counts / float(LANES)) * float(LANES), 0.0)
    sut = (r < c).astype(jnp.float32)
    offs = jnp.dot(pcounts, sut, preferred_element_type=jnp.float32,
                   precision=lax.Precision.HIGHEST)
    rank = jnp.sum(incl * ohf, axis=1, keepdims=True) - 1.0
    base = jnp.sum(offs * ohf, axis=1, keepdims=True)
    pos = (base + rank).astype(jnp.int32)

    ends = offs + pcounts
    rowj = r.astype(jnp.float32) * float(LANES)
    filled = (jnp.broadcast_to(ends, (LANES, LANES)) <= rowj) & (c < EP1)
    te = jnp.sum(filled.astype(jnp.int32), axis=1, keepdims=True)
    te = jnp.minimum(te, E - 1)

    lane = lax.broadcasted_iota(jnp.int32, (1, LANES), 1)
    pos_j = pos // LANES
    pos_l = pos - pos_j * LANES
    af = (pos_j == lane).astype(jnp.float32)
    indl = (pos_l == lane).astype(jnp.float32)
    tcol = lax.broadcasted_iota(jnp.int32, (T, 1), 0)
    thi = (tcol // 16).astype(jnp.float32)
    tlo = (tcol - (tcol // 16) * 16).astype(jnp.float32)
    g1 = gate.astype(jnp.bfloat16).astype(jnp.float32)
    g2 = gate - g1
    dn = (((0,), (0,)), ((), ()))
    hi_g = lax.dot_general(af, indl * thi, dn,
                           preferred_element_type=jnp.float32)
    lo_g = lax.dot_general(af, indl * tlo, dn,
                           preferred_element_type=jnp.float32)
    cov_g = lax.dot_general(af, indl, dn,
                            preferred_element_type=jnp.float32)
    gs1_g = lax.dot_general(af, indl * g1, dn,
                            preferred_element_type=jnp.float32)
    gs2_g = lax.dot_general(af, indl * g2, dn,
                            preferred_element_type=jnp.float32)
    ids_g = hi_g * 16.0 + lo_g + float(TRASH) * (1.0 - cov_g)
    gs_g = gs1_g + gs2_g
    ids_ref[...] = ids_g.astype(jnp.int32)
    gs_ref[...] = gs_g

    logits_ref[...] = lm
    sel_ref[...] = sel
    te_ref[...] = te


NSTREAM = 8
IPS = LANES // NSTREAM


def _gather_body(xpad_hbm, ids_hbm, xs_hbm, idx8, rows, sem):
    wid = lax.axis_index("s") * NC + lax.axis_index("c")

    def do_chunk(chunk):
        pltpu.sync_copy(ids_hbm.at[pl.ds(chunk * NSTREAM, NSTREAM)], idx8)
        copies = [
            pltpu.async_copy(xpad_hbm.at[idx8.at[i]],
                             rows.at[pl.ds(i * IPS, IPS)], sem)
            for i in range(NSTREAM)
        ]
        for cp in copies:
            cp.wait()
        pltpu.sync_copy(rows, xs_hbm.at[pl.ds(chunk * LANES, LANES)])

    do_chunk(wid)

    @pl.when(wid == 0)
    def _():
        do_chunk(jnp.int32(NT - 1))


def _scatter_body(ys_hbm, ids_hbm, out_hbm, idx8, rows, sem):
    wid = lax.axis_index("s") * NC + lax.axis_index("c")

    def do_chunk(chunk):
        pltpu.sync_copy(ids_hbm.at[pl.ds(chunk * NSTREAM, NSTREAM)], idx8)
        pltpu.sync_copy(ys_hbm.at[pl.ds(chunk * LANES, LANES)], rows)
        copies = [
            pltpu.async_copy(rows.at[pl.ds(i * IPS, IPS)],
                             out_hbm.at[idx8.at[i]], sem)
            for i in range(NSTREAM)
        ]
        for cp in copies:
            cp.wait()

    do_chunk(wid)

    @pl.when(wid == 0)
    def _():
        do_chunk(jnp.int32(NT - 1))


def _mlp_body(te_ref, x_ref, w1_ref, w2_ref, g_ref, y_ref):
    x = x_ref[...]
    h = jax.nn.gelu(jnp.dot(x, w1_ref[0], preferred_element_type=jnp.float32))
    y = jnp.dot(h, w2_ref[0], preferred_element_type=jnp.float32)
    y_ref[...] = y * g_ref[0]


def kernel(inputs, masks, W_router, W1, W2):
    x = inputs.reshape(T, D)
    wr_pad = jnp.zeros((D, LANES), jnp.float32).at[:, :EP1].set(W_router)
    masks_pad = jnp.concatenate(
        [masks, jnp.ones((T, 1), masks.dtype),
         jnp.zeros((T, LANES - EP1), masks.dtype)], axis=1)
    x_pad = jnp.concatenate([x, jnp.zeros((1, D), x.dtype)], axis=0)

    logits, sel, ids2d, gs2d, te = pl.pallas_call(
        _router_body,
        out_shape=(
            jax.ShapeDtypeStruct((T, LANES), jnp.float32),
            jax.ShapeDtypeStruct((T, 1), jnp.int32),
            jax.ShapeDtypeStruct((LANES, LANES), jnp.int32),
            jax.ShapeDtypeStruct((LANES, LANES), jnp.float32),
            jax.ShapeDtypeStruct((LANES, 1), jnp.int32),
        ),
    )(x, wr_pad, masks_pad)
    ids = ids2d.reshape(LANES * LANES)[:NTOT].reshape(NTOT // IPS, IPS)
    gs = gs2d.reshape(LANES * LANES)[:NTOT]

    mesh2 = plsc.VectorSubcoreMesh(core_axis_name="c", subcore_axis_name="s")
    gather = pl.kernel(
        _gather_body,
        out_type=jax.ShapeDtypeStruct((NTOT, D), jnp.float32),
        mesh=mesh2,
        scratch_types=[
            pltpu.VMEM((NSTREAM, IPS), jnp.int32),
            pltpu.VMEM((LANES, D), jnp.float32),
            pltpu.SemaphoreType.DMA,
        ],
    )
    xs = gather(x_pad, ids)

    grid_spec = pltpu.PrefetchScalarGridSpec(
        num_scalar_prefetch=1,
        grid=(NT,),
        in_specs=[
            pl.BlockSpec((LANES, D), lambda j, te: (j, 0)),
            pl.BlockSpec((1, D, F), lambda j, te: (te[j], 0, 0)),
            pl.BlockSpec((1, F, D), lambda j, te: (te[j], 0, 0)),
            pl.BlockSpec((1, LANES, 1), lambda j, te: (j, 0, 0)),
        ],
        out_specs=pl.BlockSpec((LANES, D), lambda j, te: (j, 0)),
    )
    ys = pl.pallas_call(
        _mlp_body,
        grid_spec=grid_spec,
        out_shape=jax.ShapeDtypeStruct((NTOT, D), jnp.float32),
    )(te.reshape(LANES), xs, W1, W2, gs.reshape(NT, LANES, 1))

    scatter = pl.kernel(
        _scatter_body,
        out_type=jax.ShapeDtypeStruct((T + 1, D), jnp.float32),
        mesh=mesh2,
        scratch_types=[
            pltpu.VMEM((NSTREAM, IPS), jnp.int32),
            pltpu.VMEM((LANES, D), jnp.float32),
            pltpu.SemaphoreType.DMA,
        ],
    )
    out = scatter(ys, ids)

    results = out[:T].reshape(inputs.shape)
    router_logits = logits[:, :EP1]
    return results, router_logits, sel


# speedup vs baseline: 1.3250x; 1.0020x over previous
import functools

import jax
import jax.numpy as jnp
from jax import lax
from jax.experimental import pallas as pl
from jax.experimental.pallas import tpu as pltpu
from jax.experimental.pallas import tpu_sc as plsc

E = 16
EP1 = E + 1
D = 768
F = 256
T = 2048
LANES = 128
NT = T // LANES + EP1
NTOT = NT * LANES
TRASH = T
NC, NS = 2, 16


def _router_body(x_ref, wr_ref, mask_ref,
                 logits_ref, sel_ref, ids_ref, gs_ref, te_ref):
    x = x_ref[...]
    lm = jnp.dot(x, wr_ref[...], preferred_element_type=jnp.float32)
    lm = lm * mask_ref[...]
    col = lax.broadcasted_iota(jnp.int32, (T, LANES), 1)
    valid = col < EP1
    lmask = jnp.where(valid, lm, jnp.float32(-1e30))
    rowmax = jnp.max(lmask, axis=1, keepdims=True)
    ex = jnp.where(valid, jnp.exp(lmask - rowmax), 0.0)
    probs = ex / jnp.sum(ex, axis=1, keepdims=True)
    maxp = jnp.max(probs, axis=1, keepdims=True)
    sel = jnp.min(jnp.where(probs == maxp, col, LANES), axis=1, keepdims=True)
    gate = jnp.where(sel == E, 0.0, maxp)
    ohf = (col == sel).astype(jnp.float32)

    r = lax.broadcasted_iota(jnp.int32, (LANES, LANES), 0)
    c = lax.broadcasted_iota(jnp.int32, (LANES, LANES), 1)
    tril = (r >= c).astype(jnp.float32)
    carry = jnp.zeros((1, LANES), jnp.float32)
    incl_rows = []
    for b in range(T // LANES):
        blk = ohf[b * LANES:(b + 1) * LANES, :]
        incl_rows.append(
            jnp.dot(tril, blk, preferred_element_type=jnp.float32) + carry)
        carry = carry + jnp.sum(blk, axis=0, keepdims=True)
    incl = jnp.concatenate(incl_rows, axis=0)
    counts = carry

    lane = lax.broadcasted_iota(jnp.int32, (1, LANES), 1)
    pcounts = jnp.where(lane < EP1,
                        jnp.ceil(counts / float(LANES)) * float(LANES), 0.0)
    sut = (r < c).astype(jnp.float32)
    offs = jnp.dot(pcounts, sut, preferred_element_type=jnp.float32,
                   precision=lax.Precision.HIGHEST)
    rank = jnp.sum(incl * ohf, axis=1, keepdims=True) - 1.0
    base = jnp.sum(offs * ohf, axis=1, keepdims=True)
    pos = (base + rank).astype(jnp.int32)

    ends = offs + pcounts
    rowj = r.astype(jnp.float32) * float(LANES)
    filled = (jnp.broadcast_to(ends, (LANES, LANES)) <= rowj) & (c < EP1)
    te = jnp.sum(filled.astype(jnp.int32), axis=1, keepdims=True)
    te = jnp.minimum(te, E - 1)

    lane = lax.broadcasted_iota(jnp.int32, (1, LANES), 1)
    pos_j = pos // LANES
    pos_l = pos - pos_j * LANES
    af = (pos_j == lane).astype(jnp.float32)
    indl = (pos_l == lane).astype(jnp.float32)
    tcol = lax.broadcasted_iota(jnp.int32, (T, 1), 0)
    thi = (tcol // 16).astype(jnp.float32)
    tlo = (tcol - (tcol // 16) * 16).astype(jnp.float32)
    g1 = gate.astype(jnp.bfloat16).astype(jnp.float32)
    g2 = gate - g1
    dn = (((0,), (0,)), ((), ()))
    hi_g = lax.dot_general(af, indl * thi, dn,
                           preferred_element_type=jnp.float32)
    lo_g = lax.dot_general(af, indl * tlo, dn,
                           preferred_element_type=jnp.float32)
    cov_g = lax.dot_general(af, indl, dn,
                            preferred_element_type=jnp.float32)
    gs1_g = lax.dot_general(af, indl * g1, dn,
                            preferred_element_type=jnp.float32)
    gs2_g = lax.dot_general(af, indl * g2, dn,
                            preferred_element_type=jnp.float32)
    ids_g = hi_g * 16.0 + lo_g + float(TRASH) * (1.0 - cov_g)
    gs_g = gs1_g + gs2_g
    ids_ref[...] = ids_g.astype(jnp.int32)
    gs_ref[...] = gs_g

    logits_ref[...] = lm
    sel_ref[...] = sel
    te_ref[...] = te


NSTREAM = 8
IPS = LANES // NSTREAM


def _gather_body(xpad_hbm, ids_hbm, xs_hbm, idx8, rows, sem):
    wid = lax.axis_index("s") * NC + lax.axis_index("c")

    def do_chunk(chunk):
        with jax.named_scope("g_ids"):
            pltpu.sync_copy(ids_hbm.at[pl.ds(chunk * NSTREAM, NSTREAM)], idx8)
        with jax.named_scope("g_gather"):
            copies = [
                pltpu.async_copy(xpad_hbm.at[idx8.at[i]],
                                 rows.at[pl.ds(i * IPS, IPS)], sem)
                for i in range(NSTREAM)
            ]
            for cp in copies:
                cp.wait()
        with jax.named_scope("g_store"):
            pltpu.sync_copy(rows, xs_hbm.at[pl.ds(chunk * LANES, LANES)])

    do_chunk(wid)

    @pl.when(wid == 0)
    def _():
        do_chunk(jnp.int32(NT - 1))


def _scatter_body(ys_hbm, ids_hbm, out_hbm, idx8, rows, sem):
    wid = lax.axis_index("s") * NC + lax.axis_index("c")

    def do_chunk(chunk):
        pltpu.sync_copy(ids_hbm.at[pl.ds(chunk * NSTREAM, NSTREAM)], idx8)
        pltpu.sync_copy(ys_hbm.at[pl.ds(chunk * LANES, LANES)], rows)
        copies = [
            pltpu.async_copy(rows.at[pl.ds(i * IPS, IPS)],
                             out_hbm.at[idx8.at[i]], sem)
            for i in range(NSTREAM)
        ]
        for cp in copies:
            cp.wait()

    do_chunk(wid)

    @pl.when(wid == 0)
    def _():
        do_chunk(jnp.int32(NT - 1))


def _mlp_body(te_ref, x_ref, w1_ref, w2_ref, g_ref, y_ref):
    x = x_ref[...]
    h = jax.nn.gelu(jnp.dot(x, w1_ref[0], preferred_element_type=jnp.float32))
    y = jnp.dot(h, w2_ref[0], preferred_element_type=jnp.float32)
    y_ref[...] = y * g_ref[0]


def kernel(inputs, masks, W_router, W1, W2):
    x = inputs.reshape(T, D)
    wr_pad = jnp.zeros((D, LANES), jnp.float32).at[:, :EP1].set(W_router)
    masks_pad = jnp.concatenate(
        [masks, jnp.ones((T, 1), masks.dtype),
         jnp.zeros((T, LANES - EP1), masks.dtype)], axis=1)
    x_pad = jnp.concatenate([x, jnp.zeros((1, D), x.dtype)], axis=0)

    logits, sel, ids2d, gs2d, te = pl.pallas_call(
        _router_body,
        out_shape=(
            jax.ShapeDtypeStruct((T, LANES), jnp.float32),
            jax.ShapeDtypeStruct((T, 1), jnp.int32),
            jax.ShapeDtypeStruct((LANES, LANES), jnp.int32),
            jax.ShapeDtypeStruct((LANES, LANES), jnp.float32),
            jax.ShapeDtypeStruct((LANES, 1), jnp.int32),
        ),
    )(x, wr_pad, masks_pad)
    ids = ids2d.reshape(LANES * LANES)[:NTOT].reshape(NTOT // IPS, IPS)
    gs = gs2d.reshape(LANES * LANES)[:NTOT]

    mesh2 = plsc.VectorSubcoreMesh(core_axis_name="c", subcore_axis_name="s")
    gather = pl.kernel(
        _gather_body,
        out_type=jax.ShapeDtypeStruct((NTOT, D), jnp.float32),
        mesh=mesh2,
        scratch_types=[
            pltpu.VMEM((NSTREAM, IPS), jnp.int32),
            pltpu.VMEM((LANES, D), jnp.float32),
            pltpu.SemaphoreType.DMA,
        ],
    )
    xs = gather(x_pad, ids)

    grid_spec = pltpu.PrefetchScalarGridSpec(
        num_scalar_prefetch=1,
        grid=(NT,),
        in_specs=[
            pl.BlockSpec((LANES, D), lambda j, te: (j, 0)),
            pl.BlockSpec((1, D, F), lambda j, te: (te[j], 0, 0)),
            pl.BlockSpec((1, F, D), lambda j, te: (te[j], 0, 0)),
            pl.BlockSpec((1, LANES, 1), lambda j, te: (j, 0, 0)),
        ],
        out_specs=pl.BlockSpec((LANES, D), lambda j, te: (j, 0)),
    )
    ys = pl.pallas_call(
        _mlp_body,
        grid_spec=grid_spec,
        out_shape=jax.ShapeDtypeStruct((NTOT, D), jnp.float32),
    )(te.reshape(LANES), xs, W1, W2, gs.reshape(NT, LANES, 1))

    scatter = pl.kernel(
        _scatter_body,
        out_type=jax.ShapeDtypeStruct((T + 1, D), jnp.float32),
        mesh=mesh2,
        scratch_types=[
            pltpu.VMEM((NSTREAM, IPS), jnp.int32),
            pltpu.VMEM((LANES, D), jnp.float32),
            pltpu.SemaphoreType.DMA,
        ],
    )
    out = scatter(ys, ids)

    results = out[:T].reshape(inputs.shape)
    router_logits = logits[:, :EP1]
    return results, router_logits, sel
